# baseline (speedup 1.0000x reference)
import functools

import jax
import jax.numpy as jnp
from jax import lax
from jax.experimental import pallas as pl
from jax.experimental.pallas import tpu as pltpu
from jax.experimental.pallas import tpu_sc as plsc

NC = 2
NS = 16
LANE = 128
BM = 400

def _mesh():
    return plsc.VectorSubcoreMesh(core_axis_name="c", subcore_axis_name="s",
                                  num_cores=NC, num_subcores=NS)


def _make_deg(NPAD, E):
    CH = 40
    EW = E // (NC * NS)
    NCHUNK = EW // CH
    RT = NPAD // NS

    out_t = tuple(jax.ShapeDtypeStruct((NPAD, 16), jnp.float32)
                  for _ in range(NC))
    scratch = [
        pltpu.VMEM((NCHUNK, CH), jnp.int32),
        pltpu.VMEM((CH, 16), jnp.float32),
        pltpu.VMEM_SHARED((NPAD, 16), jnp.float32),
    ]

    @functools.partial(pl.kernel, out_type=out_t, mesh=_mesh(),
                       scratch_types=scratch)
    def deg(dst_h, zeros_h, deg0_h, deg1_h, idx_v, ones_v, acc):
        c = lax.axis_index("c")
        s = lax.axis_index("s")
        w = s * NC + c
        pltpu.sync_copy(dst_h.at[w], idx_v)

        def fill(i, _):
            ones_v[i, :] = jnp.full((16,), 1.0, jnp.float32)
            return ()
        lax.fori_loop(0, CH, fill, ())

        pltpu.sync_copy(zeros_h, acc.at[pl.ds(s * RT, RT)])
        plsc.subcore_barrier()

        def body(j, _):
            pltpu.sync_copy(ones_v, acc.at[idx_v.at[j]], add=True)
            return ()
        lax.fori_loop(0, NCHUNK, body, ())
        plsc.subcore_barrier()

        @pl.when(c == 0)
        def _():
            pltpu.sync_copy(acc.at[pl.ds(s * RT, RT)],
                            deg0_h.at[pl.ds(s * RT, RT)])

        @pl.when(c == 1)
        def _():
            pltpu.sync_copy(acc.at[pl.ds(s * RT, RT)],
                            deg1_h.at[pl.ds(s * RT, RT)])

    return deg


def _make_agg(NPAD, E, P):
    PB = P // NC
    ET = E // NS
    CH = 96
    NCHUNK = -(-ET // CH)
    ETP = NCHUNK * CH
    RT = NPAD // NS

    out_t = tuple(jax.ShapeDtypeStruct((NPAD, LANE), jnp.float32)
                  for _ in range(P))
    scratch = [
        pltpu.VMEM((ETP,), jnp.int32),
        pltpu.VMEM((NCHUNK, CH), jnp.int32),
        pltpu.VMEM((CH, LANE), jnp.float32),
        pltpu.VMEM((CH, LANE), jnp.float32),
        pltpu.VMEM_SHARED((NPAD, LANE), jnp.float32),
        pltpu.SemaphoreType.DMA,
        pltpu.SemaphoreType.DMA,
    ]

    @functools.partial(pl.kernel, out_type=out_t, mesh=_mesh(),
                       scratch_types=scratch)
    def agg(*refs):
        hps = refs[0:P]
        src_h, dst_h, zeros_h = refs[P:P + 3]
        outs = refs[P + 3:P + 3 + P]
        src_v, dst_v, buf_a, buf_b, acc, sem_a, sem_b = refs[P + 3 + P:]

        c = lax.axis_index("c")
        s = lax.axis_index("s")
        pltpu.sync_copy(src_h.at[pl.ds(s * ETP, ETP)], src_v)
        pltpu.sync_copy(dst_h.at[s], dst_v)

        def process(p):
            hp = hps[p]

            def start(g, buf, sem):
                pltpu.async_copy(hp.at[src_v.at[pl.ds(g * CH, CH)]], buf, sem)

            def drain(g, buf, sem):
                pltpu.make_async_copy(
                    hp.at[src_v.at[pl.ds(g * CH, CH)]], buf, sem).wait()

            pltpu.sync_copy(zeros_h, acc.at[pl.ds(s * RT, RT)])
            plsc.subcore_barrier()

            start(0, buf_a, sem_a)

            def body(i, _):
                g0 = 2 * i
                pl.when(g0 + 1 < NCHUNK)(
                    lambda: start(g0 + 1, buf_b, sem_b))
                drain(g0, buf_a, sem_a)
                pltpu.sync_copy(buf_a, acc.at[dst_v.at[g0]], add=True)
                pl.when(g0 + 2 < NCHUNK)(
                    lambda: start(g0 + 2, buf_a, sem_a))

                @pl.when(g0 + 1 < NCHUNK)
                def _():
                    drain(g0 + 1, buf_b, sem_b)
                    pltpu.sync_copy(buf_b, acc.at[dst_v.at[g0 + 1]],
                                    add=True)
                return ()
            lax.fori_loop(0, (NCHUNK + 1) // 2, body, ())
            plsc.subcore_barrier()
            pltpu.sync_copy(acc.at[pl.ds(s * RT, RT)],
                            outs[p].at[pl.ds(s * RT, RT)])
            plsc.subcore_barrier()

        for p in range(P):
            pl.when(c == p // PB)(lambda p=p: process(p))

    return agg


def _dinv_from(deg0, deg1):
    return lax.rsqrt(deg0[:, 0] + deg1[:, 0] + 1.0)


def _mm0_body(x_r, w_r, d0_r, d1_r, o0, o1, o2, o3):
    dinv = _dinv_from(d0_r[...], d1_r[...])[:, None]
    h = jnp.dot(x_r[...], w_r[...], preferred_element_type=jnp.float32)
    h = h * dinv
    for k, o in enumerate((o0, o1, o2, o3)):
        o[...] = h[:, k * LANE:(k + 1) * LANE]


def _mm1_body(a0, a1, a2, a3, h0, h1, h2, h3, d0_r, d1_r, b_r, w_r, o0, o1):
    dinv = _dinv_from(d0_r[...], d1_r[...])[:, None]
    b = b_r[...]
    cols = []
    for k, (a, h) in enumerate(zip((a0, a1, a2, a3), (h0, h1, h2, h3))):
        cols.append(jax.nn.relu(dinv * (a[...] + h[...])
                                + b[:, k * LANE:(k + 1) * LANE]))
    t = jnp.concatenate(cols, axis=1)
    g = jnp.dot(t, w_r[...], preferred_element_type=jnp.float32) * dinv
    o0[...] = g[:, :LANE]
    o1[...] = g[:, LANE:]


def _fin_body(a0, a1, h0, h1, d0_r, d1_r, b_r, o_r):
    dinv = _dinv_from(d0_r[...], d1_r[...])[:, None]
    b = b_r[...]
    left = jax.nn.relu(dinv * (a0[...] + h0[...]) + b[:, :LANE])
    right = jax.nn.relu(dinv * (a1[...] + h1[...]) + b[:, LANE:])
    o_r[...] = jnp.concatenate([left, right], axis=1)


def _row_spec(width):
    return pl.BlockSpec((BM, width), lambda i: (i, 0))


def _full_spec(shape):
    return pl.BlockSpec(shape, lambda i: tuple(0 for _ in shape))


def kernel(x, edge_index, W0, b0, W1, b1):
    N, D_IN = x.shape
    E = edge_index.shape[1]
    D_HID = W0.shape[1]
    D_OUT = W1.shape[1]

    NPAD = ((N + NS * 8 - 1) // (NS * 8)) * NS * 8

    src = edge_index[0]
    dst = edge_index[1]
    ET = E // NS
    ETP = -(-ET // 96) * 96
    src_t = jnp.pad(src.reshape(NS, ET),
                    ((0, 0), (0, ETP - ET))).reshape(-1)
    dst_t = jnp.pad(dst.reshape(NS, ET), ((0, 0), (0, ETP - ET)),
                    constant_values=N).reshape(NS, ETP // 96, 96)
    dst_w = dst.reshape(NC * NS, (E // (NC * NS)) // 40, 40)
    zeros16 = jnp.zeros((NPAD // NS, 16), jnp.float32)
    zeros128 = jnp.zeros((NPAD // NS, LANE), jnp.float32)
    b0r = b0.reshape(1, D_HID)
    b1r = b1.reshape(1, D_OUT)

    deg0, deg1 = _make_deg(NPAD, E)(dst_w, zeros16)

    grid = (N // BM,)
    h0 = pl.pallas_call(
        _mm0_body,
        grid=grid,
        in_specs=[_row_spec(D_IN), _full_spec((D_IN, D_HID)),
                  _row_spec(16), _row_spec(16)],
        out_specs=[_row_spec(LANE)] * 4,
        out_shape=[jax.ShapeDtypeStruct((N, LANE), jnp.float32)] * 4,
    )(x, W0, deg0, deg1)

    agg_fn4 = _make_agg(NPAD, E, D_HID // LANE)
    a0 = agg_fn4(*h0, src_t, dst_t, zeros128)

    h1 = pl.pallas_call(
        _mm1_body,
        grid=grid,
        in_specs=[_row_spec(LANE)] * 8
        + [_row_spec(16), _row_spec(16),
           _full_spec((1, D_HID)), _full_spec((D_HID, D_OUT))],
        out_specs=[_row_spec(LANE)] * 2,
        out_shape=[jax.ShapeDtypeStruct((N, LANE), jnp.float32)] * 2,
    )(*a0, *h0, deg0, deg1, b0r, W1)

    agg_fn2 = _make_agg(NPAD, E, D_OUT // LANE)
    a1 = agg_fn2(*h1, src_t, dst_t, zeros128)

    out = pl.pallas_call(
        _fin_body,
        grid=grid,
        in_specs=[_row_spec(LANE)] * 4
        + [_row_spec(16), _row_spec(16), _full_spec((1, D_OUT))],
        out_specs=_row_spec(D_OUT),
        out_shape=jax.ShapeDtypeStruct((N, D_OUT), jnp.float32),
    )(*a1, *h1, deg0, deg1, b1r)

    return out


# speedup vs baseline: 1.2280x; 1.2280x over previous
import functools

import jax
import jax.numpy as jnp
from jax import lax
from jax.experimental import pallas as pl
from jax.experimental.pallas import tpu as pltpu
from jax.experimental.pallas import tpu_sc as plsc

NC = 2
NS = 16
LANE = 128
BM = 400

def _mesh():
    return plsc.VectorSubcoreMesh(core_axis_name="c", subcore_axis_name="s",
                                  num_cores=NC, num_subcores=NS)


def _make_deg(NPAD, E):
    CH = 40
    EW = E // (NC * NS)
    NCHUNK = EW // CH
    RT = NPAD // NS

    out_t = tuple(jax.ShapeDtypeStruct((NPAD, 16), jnp.float32)
                  for _ in range(NC))
    scratch = [
        pltpu.VMEM((NCHUNK, CH), jnp.int32),
        pltpu.VMEM((CH, 16), jnp.float32),
        pltpu.VMEM_SHARED((NPAD, 16), jnp.float32),
    ]

    @functools.partial(pl.kernel, out_type=out_t, mesh=_mesh(),
                       scratch_types=scratch)
    def deg(dst_h, zeros_h, deg0_h, deg1_h, idx_v, ones_v, acc):
        c = lax.axis_index("c")
        s = lax.axis_index("s")
        w = s * NC + c
        pltpu.sync_copy(dst_h.at[w], idx_v)

        def fill(i, _):
            ones_v[i, :] = jnp.full((16,), 1.0, jnp.float32)
            return ()
        lax.fori_loop(0, CH, fill, ())

        pltpu.sync_copy(zeros_h, acc.at[pl.ds(s * RT, RT)])
        plsc.subcore_barrier()

        def body(j, _):
            pltpu.sync_copy(ones_v, acc.at[idx_v.at[j]], add=True)
            return ()
        lax.fori_loop(0, NCHUNK, body, ())
        plsc.subcore_barrier()

        @pl.when(c == 0)
        def _():
            pltpu.sync_copy(acc.at[pl.ds(s * RT, RT)],
                            deg0_h.at[pl.ds(s * RT, RT)])

        @pl.when(c == 1)
        def _():
            pltpu.sync_copy(acc.at[pl.ds(s * RT, RT)],
                            deg1_h.at[pl.ds(s * RT, RT)])

    return deg


def _make_agg(NPAD, E, P):
    PB = P // NC
    ET = E // NS
    CH = 80
    NCHUNK = -(-ET // CH)
    ETP = NCHUNK * CH
    RT = NPAD // NS

    out_t = tuple(jax.ShapeDtypeStruct((NPAD, LANE), jnp.float32)
                  for _ in range(P))
    scratch = [
        pltpu.VMEM((ETP,), jnp.int32),
        pltpu.VMEM((NCHUNK, CH), jnp.int32),
        pltpu.VMEM((CH, LANE), jnp.float32),
        pltpu.VMEM((CH, LANE), jnp.float32),
        pltpu.VMEM_SHARED((NPAD, LANE), jnp.float32),
        pltpu.SemaphoreType.DMA,
        pltpu.SemaphoreType.DMA,
    ]

    @functools.partial(pl.kernel, out_type=out_t, mesh=_mesh(),
                       scratch_types=scratch)
    def agg(*refs):
        hps = refs[0:P]
        src_h, dst_h, zeros_h = refs[P:P + 3]
        outs = refs[P + 3:P + 3 + P]
        src_v, dst_v, buf_a, buf_b, acc, sem_a, sem_b = refs[P + 3 + P:]

        c = lax.axis_index("c")
        s = lax.axis_index("s")
        pltpu.sync_copy(src_h.at[pl.ds(s * ETP, ETP)], src_v)
        pltpu.sync_copy(dst_h.at[s], dst_v)

        def process(p):
            hp = hps[p]

            def start(g, buf, sem):
                pltpu.async_copy(hp.at[src_v.at[pl.ds(g * CH, CH)]], buf, sem)

            def drain(g, buf, sem):
                pltpu.make_async_copy(
                    hp.at[src_v.at[pl.ds(g * CH, CH)]], buf, sem).wait()

            pltpu.sync_copy(zeros_h, acc.at[pl.ds(s * RT, RT)])
            plsc.subcore_barrier()

            start(0, buf_a, sem_a)

            def body(i, _):
                g0 = 2 * i
                pl.when(g0 + 1 < NCHUNK)(
                    lambda: start(g0 + 1, buf_b, sem_b))
                drain(g0, buf_a, sem_a)
                pltpu.sync_copy(buf_a, acc.at[dst_v.at[g0]], add=True)
                pl.when(g0 + 2 < NCHUNK)(
                    lambda: start(g0 + 2, buf_a, sem_a))

                @pl.when(g0 + 1 < NCHUNK)
                def _():
                    drain(g0 + 1, buf_b, sem_b)
                    pltpu.sync_copy(buf_b, acc.at[dst_v.at[g0 + 1]],
                                    add=True)
                return ()
            lax.fori_loop(0, (NCHUNK + 1) // 2, body, ())
            plsc.subcore_barrier()
            pltpu.sync_copy(acc.at[pl.ds(s * RT, RT)],
                            outs[p].at[pl.ds(s * RT, RT)])
            plsc.subcore_barrier()

        for p in range(P):
            pl.when(c == p // PB)(lambda p=p: process(p))

    return agg


def _dinv_from(deg0, deg1):
    return lax.rsqrt(deg0[:, 0] + deg1[:, 0] + 1.0)


def _mm0_body(x_r, w_r, d0_r, d1_r, o0, o1, o2, o3):
    dinv = _dinv_from(d0_r[...], d1_r[...])[:, None]
    h = jnp.dot(x_r[...], w_r[...], preferred_element_type=jnp.float32)
    h = h * dinv
    for k, o in enumerate((o0, o1, o2, o3)):
        o[...] = h[:, k * LANE:(k + 1) * LANE]


def _mm1_body(a0, a1, a2, a3, h0, h1, h2, h3, d0_r, d1_r, b_r, w_r, o0, o1):
    dinv = _dinv_from(d0_r[...], d1_r[...])[:, None]
    b = b_r[...]
    cols = []
    for k, (a, h) in enumerate(zip((a0, a1, a2, a3), (h0, h1, h2, h3))):
        cols.append(jax.nn.relu(dinv * (a[...] + h[...])
                                + b[:, k * LANE:(k + 1) * LANE]))
    t = jnp.concatenate(cols, axis=1)
    g = jnp.dot(t, w_r[...], preferred_element_type=jnp.float32) * dinv
    o0[...] = g[:, :LANE]
    o1[...] = g[:, LANE:]


def _fin_body(a0, a1, h0, h1, d0_r, d1_r, b_r, o_r):
    dinv = _dinv_from(d0_r[...], d1_r[...])[:, None]
    b = b_r[...]
    left = jax.nn.relu(dinv * (a0[...] + h0[...]) + b[:, :LANE])
    right = jax.nn.relu(dinv * (a1[...] + h1[...]) + b[:, LANE:])
    o_r[...] = jnp.concatenate([left, right], axis=1)


def _row_spec(width):
    return pl.BlockSpec((BM, width), lambda i: (i, 0))


def _full_spec(shape):
    return pl.BlockSpec(shape, lambda i: tuple(0 for _ in shape))


def kernel(x, edge_index, W0, b0, W1, b1):
    N, D_IN = x.shape
    E = edge_index.shape[1]
    D_HID = W0.shape[1]
    D_OUT = W1.shape[1]

    NPAD = ((N + NS * 8 - 1) // (NS * 8)) * NS * 8

    src = edge_index[0]
    dst = edge_index[1]
    ET = E // NS
    ETP = -(-ET // 80) * 80
    src_t = jnp.pad(src.reshape(NS, ET),
                    ((0, 0), (0, ETP - ET))).reshape(-1)
    dst_t = jnp.pad(dst.reshape(NS, ET), ((0, 0), (0, ETP - ET)),
                    constant_values=N).reshape(NS, ETP // 80, 80)
    dst_w = dst.reshape(NC * NS, (E // (NC * NS)) // 40, 40)
    zeros16 = jnp.zeros((NPAD // NS, 16), jnp.float32)
    zeros128 = jnp.zeros((NPAD // NS, LANE), jnp.float32)
    b0r = b0.reshape(1, D_HID)
    b1r = b1.reshape(1, D_OUT)

    deg0, deg1 = _make_deg(NPAD, E)(dst_w, zeros16)

    grid = (N // BM,)
    h0 = pl.pallas_call(
        _mm0_body,
        grid=grid,
        in_specs=[_row_spec(D_IN), _full_spec((D_IN, D_HID)),
                  _row_spec(16), _row_spec(16)],
        out_specs=[_row_spec(LANE)] * 4,
        out_shape=[jax.ShapeDtypeStruct((N, LANE), jnp.float32)] * 4,
    )(x, W0, deg0, deg1)

    agg_fn4 = _make_agg(NPAD, E, D_HID // LANE)
    a0 = agg_fn4(*h0, src_t, dst_t, zeros128)

    h1 = pl.pallas_call(
        _mm1_body,
        grid=grid,
        in_specs=[_row_spec(LANE)] * 8
        + [_row_spec(16), _row_spec(16),
           _full_spec((1, D_HID)), _full_spec((D_HID, D_OUT))],
        out_specs=[_row_spec(LANE)] * 2,
        out_shape=[jax.ShapeDtypeStruct((N, LANE), jnp.float32)] * 2,
    )(*a0, *h0, deg0, deg1, b0r, W1)

    agg_fn2 = _make_agg(NPAD, E, D_OUT // LANE)
    a1 = agg_fn2(*h1, src_t, dst_t, zeros128)

    out = pl.pallas_call(
        _fin_body,
        grid=grid,
        in_specs=[_row_spec(LANE)] * 4
        + [_row_spec(16), _row_spec(16), _full_spec((1, D_OUT))],
        out_specs=_row_spec(D_OUT),
        out_shape=jax.ShapeDtypeStruct((N, D_OUT), jnp.float32),
    )(*a1, *h1, deg0, deg1, b1r)

    return out


# speedup vs baseline: 1.2938x; 1.0536x over previous
import functools

import jax
import jax.numpy as jnp
from jax import lax
from jax.experimental import pallas as pl
from jax.experimental.pallas import tpu as pltpu
from jax.experimental.pallas import tpu_sc as plsc

NC = 2
NS = 16
LANE = 128
BM = 1000

def _mesh():
    return plsc.VectorSubcoreMesh(core_axis_name="c", subcore_axis_name="s",
                                  num_cores=NC, num_subcores=NS)


def _make_deg(NPAD, E):
    CH = 40
    EW = E // (NC * NS)
    NCHUNK = EW // CH
    RT = NPAD // NS

    out_t = tuple(jax.ShapeDtypeStruct((NPAD, 16), jnp.float32)
                  for _ in range(NC))
    scratch = [
        pltpu.VMEM((NCHUNK, CH), jnp.int32),
        pltpu.VMEM((CH, 16), jnp.float32),
        pltpu.VMEM_SHARED((NPAD, 16), jnp.float32),
    ]

    @functools.partial(pl.kernel, out_type=out_t, mesh=_mesh(),
                       scratch_types=scratch)
    def deg(dst_h, zeros_h, deg0_h, deg1_h, idx_v, ones_v, acc):
        c = lax.axis_index("c")
        s = lax.axis_index("s")
        w = s * NC + c
        pltpu.sync_copy(dst_h.at[w], idx_v)

        def fill(i, _):
            ones_v[i, :] = jnp.full((16,), 1.0, jnp.float32)
            return ()
        lax.fori_loop(0, CH, fill, ())

        pltpu.sync_copy(zeros_h, acc.at[pl.ds(s * RT, RT)])
        plsc.subcore_barrier()

        def body(j, _):
            pltpu.sync_copy(ones_v, acc.at[idx_v.at[j]], add=True)
            return ()
        lax.fori_loop(0, NCHUNK, body, ())
        plsc.subcore_barrier()

        @pl.when(c == 0)
        def _():
            pltpu.sync_copy(acc.at[pl.ds(s * RT, RT)],
                            deg0_h.at[pl.ds(s * RT, RT)])

        @pl.when(c == 1)
        def _():
            pltpu.sync_copy(acc.at[pl.ds(s * RT, RT)],
                            deg1_h.at[pl.ds(s * RT, RT)])

    return deg


def _make_agg(NPAD, E, P):
    PB = P // NC
    ET = E // NS
    CH = 80
    NCHUNK = -(-ET // CH)
    ETP = NCHUNK * CH
    RT = NPAD // NS

    out_t = tuple(jax.ShapeDtypeStruct((NPAD, LANE), jnp.float32)
                  for _ in range(P))
    scratch = [
        pltpu.VMEM((ETP,), jnp.int32),
        pltpu.VMEM((NCHUNK, CH), jnp.int32),
        pltpu.VMEM((CH, LANE), jnp.float32),
        pltpu.VMEM((CH, LANE), jnp.float32),
        pltpu.VMEM_SHARED((NPAD, LANE), jnp.float32),
        pltpu.SemaphoreType.DMA,
        pltpu.SemaphoreType.DMA,
    ]

    @functools.partial(pl.kernel, out_type=out_t, mesh=_mesh(),
                       scratch_types=scratch)
    def agg(*refs):
        hps = refs[0:P]
        src_h, dst_h, zeros_h = refs[P:P + 3]
        outs = refs[P + 3:P + 3 + P]
        src_v, dst_v, buf_a, buf_b, acc, sem_a, sem_b = refs[P + 3 + P:]

        c = lax.axis_index("c")
        s = lax.axis_index("s")
        pltpu.sync_copy(src_h.at[pl.ds(s * ETP, ETP)], src_v)
        pltpu.sync_copy(dst_h.at[s], dst_v)

        def process(p):
            hp = hps[p]

            def start(g, buf, sem):
                pltpu.async_copy(hp.at[src_v.at[pl.ds(g * CH, CH)]], buf, sem)

            def drain(g, buf, sem):
                pltpu.make_async_copy(
                    hp.at[src_v.at[pl.ds(g * CH, CH)]], buf, sem).wait()

            pltpu.sync_copy(zeros_h, acc.at[pl.ds(s * RT, RT)])
            plsc.subcore_barrier()

            start(0, buf_a, sem_a)

            def body(i, _):
                g0 = 2 * i
                pl.when(g0 + 1 < NCHUNK)(
                    lambda: start(g0 + 1, buf_b, sem_b))
                drain(g0, buf_a, sem_a)
                pltpu.sync_copy(buf_a, acc.at[dst_v.at[g0]], add=True)
                pl.when(g0 + 2 < NCHUNK)(
                    lambda: start(g0 + 2, buf_a, sem_a))

                @pl.when(g0 + 1 < NCHUNK)
                def _():
                    drain(g0 + 1, buf_b, sem_b)
                    pltpu.sync_copy(buf_b, acc.at[dst_v.at[g0 + 1]],
                                    add=True)
                return ()
            lax.fori_loop(0, (NCHUNK + 1) // 2, body, ())
            plsc.subcore_barrier()
            pltpu.sync_copy(acc.at[pl.ds(s * RT, RT)],
                            outs[p].at[pl.ds(s * RT, RT)])
            plsc.subcore_barrier()

        for p in range(P):
            pl.when(c == p // PB)(lambda p=p: process(p))

    return agg


def _dinv_from(deg0, deg1):
    return lax.rsqrt(deg0[:, 0] + deg1[:, 0] + 1.0)


def _mm0_body(x_r, w_r, d0_r, d1_r, o0, o1, o2, o3):
    dinv = _dinv_from(d0_r[...], d1_r[...])[:, None]
    h = jnp.dot(x_r[...], w_r[...], preferred_element_type=jnp.float32)
    h = h * dinv
    for k, o in enumerate((o0, o1, o2, o3)):
        o[...] = h[:, k * LANE:(k + 1) * LANE]


def _mm1_body(a0, a1, a2, a3, h0, h1, h2, h3, d0_r, d1_r, b_r, w_r, o0, o1):
    dinv = _dinv_from(d0_r[...], d1_r[...])[:, None]
    b = b_r[...]
    cols = []
    for k, (a, h) in enumerate(zip((a0, a1, a2, a3), (h0, h1, h2, h3))):
        cols.append(jax.nn.relu(dinv * (a[...] + h[...])
                                + b[:, k * LANE:(k + 1) * LANE]))
    t = jnp.concatenate(cols, axis=1)
    g = jnp.dot(t, w_r[...], preferred_element_type=jnp.float32) * dinv
    o0[...] = g[:, :LANE]
    o1[...] = g[:, LANE:]


def _fin_body(a0, a1, h0, h1, d0_r, d1_r, b_r, o_r):
    dinv = _dinv_from(d0_r[...], d1_r[...])[:, None]
    b = b_r[...]
    left = jax.nn.relu(dinv * (a0[...] + h0[...]) + b[:, :LANE])
    right = jax.nn.relu(dinv * (a1[...] + h1[...]) + b[:, LANE:])
    o_r[...] = jnp.concatenate([left, right], axis=1)


def _row_spec(width):
    return pl.BlockSpec((BM, width), lambda i: (i, 0))


def _full_spec(shape):
    return pl.BlockSpec(shape, lambda i: tuple(0 for _ in shape))


def kernel(x, edge_index, W0, b0, W1, b1):
    N, D_IN = x.shape
    E = edge_index.shape[1]
    D_HID = W0.shape[1]
    D_OUT = W1.shape[1]

    NPAD = ((N + NS * 8 - 1) // (NS * 8)) * NS * 8

    src = edge_index[0]
    dst = edge_index[1]
    ET = E // NS
    ETP = -(-ET // 80) * 80
    src_t = jnp.pad(src.reshape(NS, ET),
                    ((0, 0), (0, ETP - ET))).reshape(-1)
    dst_t = jnp.pad(dst.reshape(NS, ET), ((0, 0), (0, ETP - ET)),
                    constant_values=N).reshape(NS, ETP // 80, 80)
    dst_w = dst.reshape(NC * NS, (E // (NC * NS)) // 40, 40)
    zeros16 = jnp.zeros((NPAD // NS, 16), jnp.float32)
    zeros128 = jnp.zeros((NPAD // NS, LANE), jnp.float32)
    b0r = b0.reshape(1, D_HID)
    b1r = b1.reshape(1, D_OUT)

    deg0, deg1 = _make_deg(NPAD, E)(dst_w, zeros16)

    grid = (N // BM,)
    h0 = pl.pallas_call(
        _mm0_body,
        grid=grid,
        in_specs=[_row_spec(D_IN), _full_spec((D_IN, D_HID)),
                  _row_spec(16), _row_spec(16)],
        out_specs=[_row_spec(LANE)] * 4,
        out_shape=[jax.ShapeDtypeStruct((N, LANE), jnp.float32)] * 4,
    )(x, W0, deg0, deg1)

    agg_fn4 = _make_agg(NPAD, E, D_HID // LANE)
    a0 = agg_fn4(*h0, src_t, dst_t, zeros128)

    h1 = pl.pallas_call(
        _mm1_body,
        grid=grid,
        in_specs=[_row_spec(LANE)] * 8
        + [_row_spec(16), _row_spec(16),
           _full_spec((1, D_HID)), _full_spec((D_HID, D_OUT))],
        out_specs=[_row_spec(LANE)] * 2,
        out_shape=[jax.ShapeDtypeStruct((N, LANE), jnp.float32)] * 2,
    )(*a0, *h0, deg0, deg1, b0r, W1)

    agg_fn2 = _make_agg(NPAD, E, D_OUT // LANE)
    a1 = agg_fn2(*h1, src_t, dst_t, zeros128)

    out = pl.pallas_call(
        _fin_body,
        grid=grid,
        in_specs=[_row_spec(LANE)] * 4
        + [_row_spec(16), _row_spec(16), _full_spec((1, D_OUT))],
        out_specs=_row_spec(D_OUT),
        out_shape=jax.ShapeDtypeStruct((N, D_OUT), jnp.float32),
    )(*a1, *h1, deg0, deg1, b1r)

    return out


# speedup vs baseline: 1.3024x; 1.0066x over previous
import functools

import jax
import jax.numpy as jnp
from jax import lax
from jax.experimental import pallas as pl
from jax.experimental.pallas import tpu as pltpu
from jax.experimental.pallas import tpu_sc as plsc

NC = 2
NS = 16
LANE = 128
BM = 2000

def _mesh():
    return plsc.VectorSubcoreMesh(core_axis_name="c", subcore_axis_name="s",
                                  num_cores=NC, num_subcores=NS)


def _make_deg(NPAD, E):
    CH = 40
    EW = E // (NC * NS)
    NCHUNK = EW // CH
    RT = NPAD // NS

    out_t = tuple(jax.ShapeDtypeStruct((NPAD, 16), jnp.float32)
                  for _ in range(NC))
    scratch = [
        pltpu.VMEM((NCHUNK, CH), jnp.int32),
        pltpu.VMEM((CH, 16), jnp.float32),
        pltpu.VMEM_SHARED((NPAD, 16), jnp.float32),
    ]

    @functools.partial(pl.kernel, out_type=out_t, mesh=_mesh(),
                       scratch_types=scratch)
    def deg(dst_h, zeros_h, deg0_h, deg1_h, idx_v, ones_v, acc):
        c = lax.axis_index("c")
        s = lax.axis_index("s")
        w = s * NC + c
        pltpu.sync_copy(dst_h.at[w], idx_v)

        def fill(i, _):
            ones_v[i, :] = jnp.full((16,), 1.0, jnp.float32)
            return ()
        lax.fori_loop(0, CH, fill, ())

        pltpu.sync_copy(zeros_h, acc.at[pl.ds(s * RT, RT)])
        plsc.subcore_barrier()

        def body(j, _):
            pltpu.sync_copy(ones_v, acc.at[idx_v.at[j]], add=True)
            return ()
        lax.fori_loop(0, NCHUNK, body, ())
        plsc.subcore_barrier()

        @pl.when(c == 0)
        def _():
            pltpu.sync_copy(acc.at[pl.ds(s * RT, RT)],
                            deg0_h.at[pl.ds(s * RT, RT)])

        @pl.when(c == 1)
        def _():
            pltpu.sync_copy(acc.at[pl.ds(s * RT, RT)],
                            deg1_h.at[pl.ds(s * RT, RT)])

    return deg


def _make_agg(NPAD, E, P):
    PB = P // NC
    ET = E // NS
    CH = 80
    NCHUNK = -(-ET // CH)
    ETP = NCHUNK * CH
    RT = NPAD // NS

    out_t = tuple(jax.ShapeDtypeStruct((NPAD, LANE), jnp.float32)
                  for _ in range(P))
    scratch = [
        pltpu.VMEM((ETP,), jnp.int32),
        pltpu.VMEM((NCHUNK, CH), jnp.int32),
        pltpu.VMEM((CH, LANE), jnp.float32),
        pltpu.VMEM((CH, LANE), jnp.float32),
        pltpu.VMEM_SHARED((NPAD, LANE), jnp.float32),
        pltpu.SemaphoreType.DMA,
        pltpu.SemaphoreType.DMA,
    ]

    @functools.partial(pl.kernel, out_type=out_t, mesh=_mesh(),
                       scratch_types=scratch)
    def agg(*refs):
        hps = refs[0:P]
        src_h, dst_h, zeros_h = refs[P:P + 3]
        outs = refs[P + 3:P + 3 + P]
        src_v, dst_v, buf_a, buf_b, acc, sem_a, sem_b = refs[P + 3 + P:]

        c = lax.axis_index("c")
        s = lax.axis_index("s")
        pltpu.sync_copy(src_h.at[pl.ds(s * ETP, ETP)], src_v)
        pltpu.sync_copy(dst_h.at[s], dst_v)

        def process(p):
            hp = hps[p]

            def start(g, buf, sem):
                pltpu.async_copy(hp.at[src_v.at[pl.ds(g * CH, CH)]], buf, sem)

            def drain(g, buf, sem):
                pltpu.make_async_copy(
                    hp.at[src_v.at[pl.ds(g * CH, CH)]], buf, sem).wait()

            pltpu.sync_copy(zeros_h, acc.at[pl.ds(s * RT, RT)])
            plsc.subcore_barrier()

            start(0, buf_a, sem_a)

            def body(i, _):
                g0 = 2 * i
                pl.when(g0 + 1 < NCHUNK)(
                    lambda: start(g0 + 1, buf_b, sem_b))
                drain(g0, buf_a, sem_a)
                pltpu.sync_copy(buf_a, acc.at[dst_v.at[g0]], add=True)
                pl.when(g0 + 2 < NCHUNK)(
                    lambda: start(g0 + 2, buf_a, sem_a))

                @pl.when(g0 + 1 < NCHUNK)
                def _():
                    drain(g0 + 1, buf_b, sem_b)
                    pltpu.sync_copy(buf_b, acc.at[dst_v.at[g0 + 1]],
                                    add=True)
                return ()
            lax.fori_loop(0, (NCHUNK + 1) // 2, body, ())
            plsc.subcore_barrier()
            pltpu.sync_copy(acc.at[pl.ds(s * RT, RT)],
                            outs[p].at[pl.ds(s * RT, RT)])
            plsc.subcore_barrier()

        for p in range(P):
            pl.when(c == p // PB)(lambda p=p: process(p))

    return agg


def _dinv_from(deg0, deg1):
    return lax.rsqrt(deg0[:, 0] + deg1[:, 0] + 1.0)


def _mm0_body(x_r, w_r, d0_r, d1_r, o0, o1, o2, o3):
    dinv = _dinv_from(d0_r[...], d1_r[...])[:, None]
    h = jnp.dot(x_r[...], w_r[...], preferred_element_type=jnp.float32)
    h = h * dinv
    for k, o in enumerate((o0, o1, o2, o3)):
        o[...] = h[:, k * LANE:(k + 1) * LANE]


def _mm1_body(a0, a1, a2, a3, h0, h1, h2, h3, d0_r, d1_r, b_r, w_r, o0, o1):
    dinv = _dinv_from(d0_r[...], d1_r[...])[:, None]
    b = b_r[...]
    cols = []
    for k, (a, h) in enumerate(zip((a0, a1, a2, a3), (h0, h1, h2, h3))):
        cols.append(jax.nn.relu(dinv * (a[...] + h[...])
                                + b[:, k * LANE:(k + 1) * LANE]))
    t = jnp.concatenate(cols, axis=1)
    g = jnp.dot(t, w_r[...], preferred_element_type=jnp.float32) * dinv
    o0[...] = g[:, :LANE]
    o1[...] = g[:, LANE:]


def _fin_body(a0, a1, h0, h1, d0_r, d1_r, b_r, o_r):
    dinv = _dinv_from(d0_r[...], d1_r[...])[:, None]
    b = b_r[...]
    left = jax.nn.relu(dinv * (a0[...] + h0[...]) + b[:, :LANE])
    right = jax.nn.relu(dinv * (a1[...] + h1[...]) + b[:, LANE:])
    o_r[...] = jnp.concatenate([left, right], axis=1)


def _row_spec(width):
    return pl.BlockSpec((BM, width), lambda i: (i, 0))


def _full_spec(shape):
    return pl.BlockSpec(shape, lambda i: tuple(0 for _ in shape))


def kernel(x, edge_index, W0, b0, W1, b1):
    N, D_IN = x.shape
    E = edge_index.shape[1]
    D_HID = W0.shape[1]
    D_OUT = W1.shape[1]

    NPAD = ((N + NS * 8 - 1) // (NS * 8)) * NS * 8

    src = edge_index[0]
    dst = edge_index[1]
    ET = E // NS
    ETP = -(-ET // 80) * 80
    src_t = jnp.pad(src.reshape(NS, ET),
                    ((0, 0), (0, ETP - ET))).reshape(-1)
    dst_t = jnp.pad(dst.reshape(NS, ET), ((0, 0), (0, ETP - ET)),
                    constant_values=N).reshape(NS, ETP // 80, 80)
    dst_w = dst.reshape(NC * NS, (E // (NC * NS)) // 40, 40)
    zeros16 = jnp.zeros((NPAD // NS, 16), jnp.float32)
    zeros128 = jnp.zeros((NPAD // NS, LANE), jnp.float32)
    b0r = b0.reshape(1, D_HID)
    b1r = b1.reshape(1, D_OUT)

    deg0, deg1 = _make_deg(NPAD, E)(dst_w, zeros16)

    grid = (N // BM,)
    h0 = pl.pallas_call(
        _mm0_body,
        grid=grid,
        in_specs=[_row_spec(D_IN), _full_spec((D_IN, D_HID)),
                  _row_spec(16), _row_spec(16)],
        out_specs=[_row_spec(LANE)] * 4,
        out_shape=[jax.ShapeDtypeStruct((N, LANE), jnp.float32)] * 4,
    )(x, W0, deg0, deg1)

    agg_fn4 = _make_agg(NPAD, E, D_HID // LANE)
    a0 = agg_fn4(*h0, src_t, dst_t, zeros128)

    h1 = pl.pallas_call(
        _mm1_body,
        grid=grid,
        in_specs=[_row_spec(LANE)] * 8
        + [_row_spec(16), _row_spec(16),
           _full_spec((1, D_HID)), _full_spec((D_HID, D_OUT))],
        out_specs=[_row_spec(LANE)] * 2,
        out_shape=[jax.ShapeDtypeStruct((N, LANE), jnp.float32)] * 2,
    )(*a0, *h0, deg0, deg1, b0r, W1)

    agg_fn2 = _make_agg(NPAD, E, D_OUT // LANE)
    a1 = agg_fn2(*h1, src_t, dst_t, zeros128)

    out = pl.pallas_call(
        _fin_body,
        grid=grid,
        in_specs=[_row_spec(LANE)] * 4
        + [_row_spec(16), _row_spec(16), _full_spec((1, D_OUT))],
        out_specs=_row_spec(D_OUT),
        out_shape=jax.ShapeDtypeStruct((N, D_OUT), jnp.float32),
    )(*a1, *h1, deg0, deg1, b1r)

    return out
